# Initial kernel scaffold; baseline (speedup 1.0000x reference)
import jax
import jax.numpy as jnp
from jax.experimental import pallas as pl


def kernel(x, W):
    raise NotImplementedError("write your pallas kernel here")


# speedup vs baseline: 3.1275x; 3.1275x over previous
import functools

import jax
import jax.numpy as jnp
from jax.experimental import pallas as pl
from jax.experimental.pallas import tpu as pltpu

T = 16384
D = 2048
E = 64
G = 8
EPG = 8
TKG = 2
S = 4
K = 8

BLK = 512
NBLK = T // BLK


def _router_kernel(x_ref, wt_ref, w_ref, id_ref, aux_ref, hist_ref, psum_ref):
    i = pl.program_id(0)

    logits = jnp.dot(x_ref[...], wt_ref[...], preferred_element_type=jnp.float32)
    lt = logits.T

    m = jnp.max(lt, axis=0, keepdims=True)
    ex = jnp.exp(lt - m)
    p = ex / jnp.sum(ex, axis=0, keepdims=True)

    idx8 = jax.lax.broadcasted_iota(jnp.int32, (EPG, BLK), 0)

    m1s, m2s, i1s, i2s, gss = [], [], [], [], []
    for g in range(G):
        v = p[g * EPG:(g + 1) * EPG, :]
        m1 = jnp.max(v, axis=0, keepdims=True)
        i1 = jnp.min(jnp.where(v == m1, idx8, EPG), axis=0, keepdims=True)
        v2 = jnp.where(idx8 == i1, -1.0, v)
        m2 = jnp.max(v2, axis=0, keepdims=True)
        i2 = jnp.min(jnp.where(v2 == m2, idx8, EPG), axis=0, keepdims=True)
        m1s.append(m1)
        m2s.append(m2)
        i1s.append(i1)
        i2s.append(i2)
        gss.append(m1 + m2)

    m1s = jnp.concatenate(m1s, axis=0)
    m2s = jnp.concatenate(m2s, axis=0)
    i1s = jnp.concatenate(i1s, axis=0)
    i2s = jnp.concatenate(i2s, axis=0)
    gs = jnp.concatenate(gss, axis=0)

    idxg = jax.lax.broadcasted_iota(jnp.int32, (G, BLK), 0)

    rows_w, rows_id, denom = [], [], jnp.float32(1e-9)
    for s in range(S):
        gmax = jnp.max(gs, axis=0, keepdims=True)
        sel = jnp.min(jnp.where(gs == gmax, idxg, G), axis=0, keepdims=True)
        gs = jnp.where(idxg == sel, -1.0, gs)
        onsel = idxg == sel
        m1_s = jnp.sum(jnp.where(onsel, m1s, 0.0), axis=0, keepdims=True)
        m2_s = jnp.sum(jnp.where(onsel, m2s, 0.0), axis=0, keepdims=True)
        i1_s = jnp.sum(jnp.where(onsel, i1s, 0), axis=0, keepdims=True)
        i2_s = jnp.sum(jnp.where(onsel, i2s, 0), axis=0, keepdims=True)
        rows_w.extend([m1_s, m2_s])
        rows_id.extend([sel * EPG + i1_s, sel * EPG + i2_s])
        denom = denom + gmax

    wt_t = jnp.concatenate(rows_w, axis=0) / denom
    ids_t = jnp.concatenate(rows_id, axis=0)

    w_ref[...] = wt_t.T
    id_ref[...] = ids_t.T

    top1 = ids_t[0:1, :]
    one_hot = (jax.lax.broadcasted_iota(jnp.int32, (E, BLK), 0) == top1).astype(jnp.float32)

    @pl.when(i == 0)
    def _init():
        hist_ref[...] = jnp.zeros_like(hist_ref)
        psum_ref[...] = jnp.zeros_like(psum_ref)

    hist_ref[...] += jnp.sum(one_hot, axis=1, keepdims=True)
    psum_ref[...] += jnp.sum(p, axis=1, keepdims=True)

    @pl.when(i == NBLK - 1)
    def _fin():
        aux_ref[...] = (jnp.sum(hist_ref[...] * psum_ref[...])
                        * (float(E) / (float(T) * float(T)))).reshape(1, 1)


@jax.jit
def kernel(x, W):
    wt = W.T
    w_out, id_out, aux = pl.pallas_call(
        _router_kernel,
        grid=(NBLK,),
        in_specs=[
            pl.BlockSpec((BLK, D), lambda i: (i, 0)),
            pl.BlockSpec((D, E), lambda i: (0, 0)),
        ],
        out_specs=[
            pl.BlockSpec((BLK, K), lambda i: (i, 0)),
            pl.BlockSpec((BLK, K), lambda i: (i, 0)),
            pl.BlockSpec((1, 1), lambda i: (0, 0)),
        ],
        out_shape=[
            jax.ShapeDtypeStruct((T, K), jnp.float32),
            jax.ShapeDtypeStruct((T, K), jnp.int32),
            jax.ShapeDtypeStruct((1, 1), jnp.float32),
        ],
        scratch_shapes=[
            pltpu.VMEM((E, 1), jnp.float32),
            pltpu.VMEM((E, 1), jnp.float32),
        ],
    )(x, wt)
    return w_out, id_out, aux.reshape(())


# speedup vs baseline: 3.2128x; 1.0273x over previous
import functools

import jax
import jax.numpy as jnp
from jax.experimental import pallas as pl
from jax.experimental.pallas import tpu as pltpu

T = 16384
D = 2048
E = 64
G = 8
EPG = 8
TKG = 2
S = 4
K = 8

BLK = 512
NBLK = T // BLK


def _router_kernel(x_ref, wt_ref, w_ref, id_ref, aux_ref, hist_ref, psum_ref):
    i = pl.program_id(0)

    lt = jax.lax.dot_general(
        wt_ref[...], x_ref[...],
        dimension_numbers=(((1,), (1,)), ((), ())),
        preferred_element_type=jnp.float32,
    )

    m = jnp.max(lt, axis=0, keepdims=True)
    ex = jnp.exp(lt - m)
    p = ex / jnp.sum(ex, axis=0, keepdims=True)

    idx8 = jax.lax.broadcasted_iota(jnp.int32, (EPG, BLK), 0)

    m1s, m2s, i1s, i2s, gss = [], [], [], [], []
    for g in range(G):
        v = p[g * EPG:(g + 1) * EPG, :]
        m1 = jnp.max(v, axis=0, keepdims=True)
        i1 = jnp.min(jnp.where(v == m1, idx8, EPG), axis=0, keepdims=True)
        v2 = jnp.where(idx8 == i1, -1.0, v)
        m2 = jnp.max(v2, axis=0, keepdims=True)
        i2 = jnp.min(jnp.where(v2 == m2, idx8, EPG), axis=0, keepdims=True)
        m1s.append(m1)
        m2s.append(m2)
        i1s.append(i1)
        i2s.append(i2)
        gss.append(m1 + m2)

    m1s = jnp.concatenate(m1s, axis=0)
    m2s = jnp.concatenate(m2s, axis=0)
    i1s = jnp.concatenate(i1s, axis=0)
    i2s = jnp.concatenate(i2s, axis=0)
    gs = jnp.concatenate(gss, axis=0)

    idxg = jax.lax.broadcasted_iota(jnp.int32, (G, BLK), 0)

    rows_w, rows_id, denom = [], [], jnp.float32(1e-9)
    for s in range(S):
        gmax = jnp.max(gs, axis=0, keepdims=True)
        sel = jnp.min(jnp.where(gs == gmax, idxg, G), axis=0, keepdims=True)
        gs = jnp.where(idxg == sel, -1.0, gs)
        onsel = idxg == sel
        m1_s = jnp.sum(jnp.where(onsel, m1s, 0.0), axis=0, keepdims=True)
        m2_s = jnp.sum(jnp.where(onsel, m2s, 0.0), axis=0, keepdims=True)
        i1_s = jnp.sum(jnp.where(onsel, i1s, 0), axis=0, keepdims=True)
        i2_s = jnp.sum(jnp.where(onsel, i2s, 0), axis=0, keepdims=True)
        rows_w.extend([m1_s, m2_s])
        rows_id.extend([sel * EPG + i1_s, sel * EPG + i2_s])
        denom = denom + gmax

    wt_t = jnp.concatenate(rows_w, axis=0) / denom
    ids_t = jnp.concatenate(rows_id, axis=0)

    w_ref[...] = wt_t.T
    id_ref[...] = ids_t.T

    top1 = ids_t[0:1, :]
    one_hot = (jax.lax.broadcasted_iota(jnp.int32, (E, BLK), 0) == top1).astype(jnp.float32)

    @pl.when(i == 0)
    def _init():
        hist_ref[...] = jnp.zeros_like(hist_ref)
        psum_ref[...] = jnp.zeros_like(psum_ref)

    hist_ref[...] += jnp.sum(one_hot, axis=1, keepdims=True)
    psum_ref[...] += jnp.sum(p, axis=1, keepdims=True)

    @pl.when(i == NBLK - 1)
    def _fin():
        aux_ref[...] = (jnp.sum(hist_ref[...] * psum_ref[...])
                        * (float(E) / (float(T) * float(T)))).reshape(1, 1)


@jax.jit
def kernel(x, W):
    w_out, id_out, aux = pl.pallas_call(
        _router_kernel,
        grid=(NBLK,),
        in_specs=[
            pl.BlockSpec((BLK, D), lambda i: (i, 0)),
            pl.BlockSpec((E, D), lambda i: (0, 0)),
        ],
        out_specs=[
            pl.BlockSpec((BLK, K), lambda i: (i, 0)),
            pl.BlockSpec((BLK, K), lambda i: (i, 0)),
            pl.BlockSpec((1, 1), lambda i: (0, 0)),
        ],
        out_shape=[
            jax.ShapeDtypeStruct((T, K), jnp.float32),
            jax.ShapeDtypeStruct((T, K), jnp.int32),
            jax.ShapeDtypeStruct((1, 1), jnp.float32),
        ],
        scratch_shapes=[
            pltpu.VMEM((E, 1), jnp.float32),
            pltpu.VMEM((E, 1), jnp.float32),
        ],
    )(x, W)
    return w_out, id_out, aux.reshape(())


# speedup vs baseline: 3.7700x; 1.1734x over previous
import functools

import jax
import jax.numpy as jnp
from jax.experimental import pallas as pl
from jax.experimental.pallas import tpu as pltpu

T = 16384
D = 2048
E = 64
G = 8
EPG = 8
TKG = 2
S = 4
K = 8

BLK = 1024
NBLK = T // BLK


def _router_kernel(x_ref, wt_ref, w_ref, id_ref, aux_ref, hist_ref, psum_ref):
    i = pl.program_id(0)

    lt = jax.lax.dot_general(
        wt_ref[...], x_ref[...],
        dimension_numbers=(((1,), (1,)), ((), ())),
        preferred_element_type=jnp.float32,
    )

    m = jnp.max(lt, axis=0, keepdims=True)
    ex = jnp.exp(lt - m)
    p = ex / jnp.sum(ex, axis=0, keepdims=True)

    idx8 = jax.lax.broadcasted_iota(jnp.int32, (EPG, BLK), 0)

    m1s, m2s, i1s, i2s, gss = [], [], [], [], []
    for g in range(G):
        v = p[g * EPG:(g + 1) * EPG, :]
        m1 = jnp.max(v, axis=0, keepdims=True)
        i1 = jnp.min(jnp.where(v == m1, idx8, EPG), axis=0, keepdims=True)
        v2 = jnp.where(idx8 == i1, -1.0, v)
        m2 = jnp.max(v2, axis=0, keepdims=True)
        i2 = jnp.min(jnp.where(v2 == m2, idx8, EPG), axis=0, keepdims=True)
        m1s.append(m1)
        m2s.append(m2)
        i1s.append(i1)
        i2s.append(i2)
        gss.append(m1 + m2)

    m1s = jnp.concatenate(m1s, axis=0)
    m2s = jnp.concatenate(m2s, axis=0)
    i1s = jnp.concatenate(i1s, axis=0)
    i2s = jnp.concatenate(i2s, axis=0)
    gs = jnp.concatenate(gss, axis=0)

    idxg = jax.lax.broadcasted_iota(jnp.int32, (G, BLK), 0)

    rows_w, rows_id, denom = [], [], jnp.float32(1e-9)
    for s in range(S):
        gmax = jnp.max(gs, axis=0, keepdims=True)
        sel = jnp.min(jnp.where(gs == gmax, idxg, G), axis=0, keepdims=True)
        gs = jnp.where(idxg == sel, -1.0, gs)
        onsel = idxg == sel
        m1_s = jnp.sum(jnp.where(onsel, m1s, 0.0), axis=0, keepdims=True)
        m2_s = jnp.sum(jnp.where(onsel, m2s, 0.0), axis=0, keepdims=True)
        i1_s = jnp.sum(jnp.where(onsel, i1s, 0), axis=0, keepdims=True)
        i2_s = jnp.sum(jnp.where(onsel, i2s, 0), axis=0, keepdims=True)
        rows_w.extend([m1_s, m2_s])
        rows_id.extend([sel * EPG + i1_s, sel * EPG + i2_s])
        denom = denom + gmax

    wt_t = jnp.concatenate(rows_w, axis=0) / denom
    ids_t = jnp.concatenate(rows_id, axis=0)

    w_ref[...] = wt_t.T
    id_ref[...] = ids_t.T

    top1 = ids_t[0:1, :]
    one_hot = (jax.lax.broadcasted_iota(jnp.int32, (E, BLK), 0) == top1).astype(jnp.float32)

    @pl.when(i == 0)
    def _init():
        hist_ref[...] = jnp.zeros_like(hist_ref)
        psum_ref[...] = jnp.zeros_like(psum_ref)

    hist_ref[...] += jnp.sum(one_hot, axis=1, keepdims=True)
    psum_ref[...] += jnp.sum(p, axis=1, keepdims=True)

    @pl.when(i == NBLK - 1)
    def _fin():
        aux_ref[...] = (jnp.sum(hist_ref[...] * psum_ref[...])
                        * (float(E) / (float(T) * float(T)))).reshape(1, 1)


@jax.jit
def kernel(x, W):
    w_out, id_out, aux = pl.pallas_call(
        _router_kernel,
        grid=(NBLK,),
        in_specs=[
            pl.BlockSpec((BLK, D), lambda i: (i, 0)),
            pl.BlockSpec((E, D), lambda i: (0, 0)),
        ],
        out_specs=[
            pl.BlockSpec((BLK, K), lambda i: (i, 0)),
            pl.BlockSpec((BLK, K), lambda i: (i, 0)),
            pl.BlockSpec((1, 1), lambda i: (0, 0)),
        ],
        out_shape=[
            jax.ShapeDtypeStruct((T, K), jnp.float32),
            jax.ShapeDtypeStruct((T, K), jnp.int32),
            jax.ShapeDtypeStruct((1, 1), jnp.float32),
        ],
        scratch_shapes=[
            pltpu.VMEM((E, 1), jnp.float32),
            pltpu.VMEM((E, 1), jnp.float32),
        ],
    )(x, W)
    return w_out, id_out, aux.reshape(())


# speedup vs baseline: 3.9707x; 1.0532x over previous
import functools

import jax
import jax.numpy as jnp
from jax.experimental import pallas as pl
from jax.experimental.pallas import tpu as pltpu

T = 16384
D = 2048
E = 64
G = 8
EPG = 8
TKG = 2
S = 4
K = 8

BLK = 2048
NBLK = T // BLK


def _router_kernel(x_ref, wt_ref, w_ref, id_ref, aux_ref, hist_ref, psum_ref):
    i = pl.program_id(0)

    lt = jax.lax.dot_general(
        wt_ref[...], x_ref[...],
        dimension_numbers=(((1,), (1,)), ((), ())),
        preferred_element_type=jnp.float32,
    )

    m = jnp.max(lt, axis=0, keepdims=True)
    ex = jnp.exp(lt - m)
    p = ex / jnp.sum(ex, axis=0, keepdims=True)

    idx8 = jax.lax.broadcasted_iota(jnp.int32, (EPG, BLK), 0)

    m1s, m2s, i1s, i2s, gss = [], [], [], [], []
    for g in range(G):
        v = p[g * EPG:(g + 1) * EPG, :]
        m1 = jnp.max(v, axis=0, keepdims=True)
        i1 = jnp.min(jnp.where(v == m1, idx8, EPG), axis=0, keepdims=True)
        v2 = jnp.where(idx8 == i1, -1.0, v)
        m2 = jnp.max(v2, axis=0, keepdims=True)
        i2 = jnp.min(jnp.where(v2 == m2, idx8, EPG), axis=0, keepdims=True)
        m1s.append(m1)
        m2s.append(m2)
        i1s.append(i1)
        i2s.append(i2)
        gss.append(m1 + m2)

    m1s = jnp.concatenate(m1s, axis=0)
    m2s = jnp.concatenate(m2s, axis=0)
    i1s = jnp.concatenate(i1s, axis=0)
    i2s = jnp.concatenate(i2s, axis=0)
    gs = jnp.concatenate(gss, axis=0)

    idxg = jax.lax.broadcasted_iota(jnp.int32, (G, BLK), 0)

    rows_w, rows_id, denom = [], [], jnp.float32(1e-9)
    for s in range(S):
        gmax = jnp.max(gs, axis=0, keepdims=True)
        sel = jnp.min(jnp.where(gs == gmax, idxg, G), axis=0, keepdims=True)
        gs = jnp.where(idxg == sel, -1.0, gs)
        onsel = idxg == sel
        m1_s = jnp.sum(jnp.where(onsel, m1s, 0.0), axis=0, keepdims=True)
        m2_s = jnp.sum(jnp.where(onsel, m2s, 0.0), axis=0, keepdims=True)
        i1_s = jnp.sum(jnp.where(onsel, i1s, 0), axis=0, keepdims=True)
        i2_s = jnp.sum(jnp.where(onsel, i2s, 0), axis=0, keepdims=True)
        rows_w.extend([m1_s, m2_s])
        rows_id.extend([sel * EPG + i1_s, sel * EPG + i2_s])
        denom = denom + gmax

    wt_t = jnp.concatenate(rows_w, axis=0) / denom
    ids_t = jnp.concatenate(rows_id, axis=0)

    w_ref[...] = wt_t.T
    id_ref[...] = ids_t.T

    top1 = ids_t[0:1, :]
    one_hot = (jax.lax.broadcasted_iota(jnp.int32, (E, BLK), 0) == top1).astype(jnp.float32)

    @pl.when(i == 0)
    def _init():
        hist_ref[...] = jnp.zeros_like(hist_ref)
        psum_ref[...] = jnp.zeros_like(psum_ref)

    hist_ref[...] += jnp.sum(one_hot, axis=1, keepdims=True)
    psum_ref[...] += jnp.sum(p, axis=1, keepdims=True)

    @pl.when(i == NBLK - 1)
    def _fin():
        aux_ref[...] = (jnp.sum(hist_ref[...] * psum_ref[...])
                        * (float(E) / (float(T) * float(T)))).reshape(1, 1)


@jax.jit
def kernel(x, W):
    w_out, id_out, aux = pl.pallas_call(
        _router_kernel,
        grid=(NBLK,),
        in_specs=[
            pl.BlockSpec((BLK, D), lambda i: (i, 0)),
            pl.BlockSpec((E, D), lambda i: (0, 0)),
        ],
        out_specs=[
            pl.BlockSpec((BLK, K), lambda i: (i, 0)),
            pl.BlockSpec((BLK, K), lambda i: (i, 0)),
            pl.BlockSpec((1, 1), lambda i: (0, 0)),
        ],
        out_shape=[
            jax.ShapeDtypeStruct((T, K), jnp.float32),
            jax.ShapeDtypeStruct((T, K), jnp.int32),
            jax.ShapeDtypeStruct((1, 1), jnp.float32),
        ],
        scratch_shapes=[
            pltpu.VMEM((E, 1), jnp.float32),
            pltpu.VMEM((E, 1), jnp.float32),
        ],
    )(x, W)
    return w_out, id_out, aux.reshape(())


# speedup vs baseline: 4.0094x; 1.0098x over previous
import functools

import jax
import jax.numpy as jnp
from jax.experimental import pallas as pl
from jax.experimental.pallas import tpu as pltpu

T = 16384
D = 2048
E = 64
G = 8
EPG = 8
TKG = 2
S = 4
K = 8

BLK = 2048
NBLK = T // BLK


def _router_kernel(x_ref, wt_ref, w_ref, id_ref, aux_ref, hist_ref, psum_ref):
    i = pl.program_id(0)

    lt = jax.lax.dot_general(
        wt_ref[...], x_ref[...],
        dimension_numbers=(((1,), (1,)), ((), ())),
        preferred_element_type=jnp.float32,
    )

    m = jnp.max(lt, axis=0, keepdims=True)
    ex = jnp.exp(lt - m)
    z = jnp.sum(ex, axis=0, keepdims=True)
    p = ex * (1.0 / z)

    bc = jax.lax.bitcast_convert_type

    iota64 = jax.lax.broadcasted_iota(jnp.int32, (E, BLK), 0)
    keys = bc((bc(p, jnp.int32) & -64) | (63 - iota64), jnp.float32)

    k1r, k2r, gkr = [], [], []
    for g in range(G):
        kg = keys[g * EPG:(g + 1) * EPG, :]
        k1 = jnp.max(kg, axis=0, keepdims=True)
        k2 = jnp.max(jnp.where(kg == k1, -1.0, kg), axis=0, keepdims=True)
        gs = bc(bc(k1, jnp.int32) & -64, jnp.float32) + bc(bc(k2, jnp.int32) & -64, jnp.float32)
        gkr.append(bc((bc(gs, jnp.int32) & -8) | (7 - g), jnp.float32))
        k1r.append(k1)
        k2r.append(k2)
    k1s = jnp.concatenate(k1r, axis=0)
    k2s = jnp.concatenate(k2r, axis=0)
    gk = jnp.concatenate(gkr, axis=0)

    rank = jnp.zeros((G, BLK), jnp.int32)
    for h in range(G):
        rank = rank + (gk < gk[h:h + 1, :]).astype(jnp.int32)

    rows_w, rows_id, denom = [], [], jnp.float32(1e-9)
    for s in range(S):
        onsel = rank == s
        b1 = bc(jnp.sum(jnp.where(onsel, k1s, 0.0), axis=0, keepdims=True), jnp.int32)
        b2 = bc(jnp.sum(jnp.where(onsel, k2s, 0.0), axis=0, keepdims=True), jnp.int32)
        v1 = bc(b1 & -64, jnp.float32)
        v2 = bc(b2 & -64, jnp.float32)
        rows_w.extend([v1, v2])
        rows_id.extend([63 - (b1 & 63), 63 - (b2 & 63)])
        denom = denom + v1 + v2

    wt_t = jnp.concatenate(rows_w, axis=0) / denom
    ids_t = jnp.concatenate(rows_id, axis=0)

    w_ref[...] = wt_t.T
    id_ref[...] = ids_t.T

    top1 = ids_t[0:1, :]
    one_hot = (jax.lax.broadcasted_iota(jnp.int32, (E, BLK), 0) == top1).astype(jnp.float32)

    @pl.when(i == 0)
    def _init():
        hist_ref[...] = jnp.zeros_like(hist_ref)
        psum_ref[...] = jnp.zeros_like(psum_ref)

    hist_ref[...] += jnp.sum(one_hot, axis=1, keepdims=True)
    psum_ref[...] += jnp.sum(p, axis=1, keepdims=True)

    @pl.when(i == NBLK - 1)
    def _fin():
        aux_ref[...] = (jnp.sum(hist_ref[...] * psum_ref[...])
                        * (float(E) / (float(T) * float(T)))).reshape(1, 1)


@jax.jit
def kernel(x, W):
    w_out, id_out, aux = pl.pallas_call(
        _router_kernel,
        grid=(NBLK,),
        in_specs=[
            pl.BlockSpec((BLK, D), lambda i: (i, 0)),
            pl.BlockSpec((E, D), lambda i: (0, 0)),
        ],
        out_specs=[
            pl.BlockSpec((BLK, K), lambda i: (i, 0)),
            pl.BlockSpec((BLK, K), lambda i: (i, 0)),
            pl.BlockSpec((1, 1), lambda i: (0, 0)),
        ],
        out_shape=[
            jax.ShapeDtypeStruct((T, K), jnp.float32),
            jax.ShapeDtypeStruct((T, K), jnp.int32),
            jax.ShapeDtypeStruct((1, 1), jnp.float32),
        ],
        scratch_shapes=[
            pltpu.VMEM((E, 1), jnp.float32),
            pltpu.VMEM((E, 1), jnp.float32),
        ],
    )(x, W)
    return w_out, id_out, aux.reshape(())
